# Initial kernel scaffold; baseline (speedup 1.0000x reference)
import functools

import jax
import jax.numpy as jnp
from jax import lax
from jax.experimental import pallas as pl
from jax.experimental.pallas import tpu as pltpu
from jax.experimental.pallas import tpu_sc as plsc

_NUM_CORES = 2
_NUM_SUBCORES = 16
_NW = _NUM_CORES * _NUM_SUBCORES
_CHUNK = 32
_LANES = 16


def _pe_add_body(x_hbm, len_hbm, pe_hbm, out_hbm, len_v, idx_v, buf, sem,
                 *, rows_per_w, seq, chunks):
  wid = lax.axis_index("s") * _NUM_CORES + lax.axis_index("c")
  base = wid * rows_per_w
  b = base // seq
  s0 = base % seq

  pltpu.sync_copy(len_hbm, len_v)
  len_vec = plsc.load_gather(len_v, [jnp.full((_LANES,), b, jnp.int32)])

  def chunk_body(g, _):
    row0 = base + g * _CHUNK
    pos0 = s0 + g * _CHUNK + 1
    for j in range(_CHUNK // _LANES):
      pos = pos0 + j * _LANES + lax.iota(jnp.int32, _LANES)
      idx = jnp.where(pos <= len_vec, pos, 0)
      idx_v[pl.ds(j * _LANES, _LANES)] = idx
    pltpu.sync_copy(x_hbm.at[pl.ds(row0, _CHUNK)], buf)
    pltpu.async_copy(pe_hbm.at[idx_v], buf, sem, add=True).wait()
    pltpu.sync_copy(buf, out_hbm.at[pl.ds(row0, _CHUNK)])
    return 0

  lax.fori_loop(0, chunks, chunk_body, 0)


def kernel(x, lengths, pe_weight):
  n_batch, n_seq, d_emb = x.shape
  total_rows = n_batch * n_seq
  rows_per_w = total_rows // _NW
  chunks = rows_per_w // _CHUNK

  xf = x.reshape(total_rows, d_emb)
  lengths = lengths.astype(jnp.int32)

  mesh = plsc.VectorSubcoreMesh(core_axis_name="c", subcore_axis_name="s")
  body = functools.partial(
      _pe_add_body, rows_per_w=rows_per_w, seq=n_seq, chunks=chunks)
  out = pl.kernel(
      body,
      out_type=jax.ShapeDtypeStruct((total_rows, d_emb), jnp.float32),
      mesh=mesh,
      scratch_types=[
          pltpu.VMEM((n_batch,), jnp.int32),
          pltpu.VMEM((_CHUNK,), jnp.int32),
          pltpu.VMEM((_CHUNK, d_emb), jnp.float32),
          pltpu.SemaphoreType.DMA,
      ],
  )(xf, lengths, pe_weight)
  return out.reshape(n_batch, n_seq, d_emb)


# speedup vs baseline: 1.0232x; 1.0232x over previous
import functools

import jax
import jax.numpy as jnp
from jax import lax
from jax.experimental import pallas as pl
from jax.experimental.pallas import tpu as pltpu
from jax.experimental.pallas import tpu_sc as plsc

_NUM_CORES = 2
_NUM_SUBCORES = 16
_NW = _NUM_CORES * _NUM_SUBCORES
_CHUNK = 32
_LANES = 16


def _pe_add_body(x_hbm, len_hbm, pe_hbm, out_hbm, len_v, idx_v, xbuf, pbuf,
                 sem, *, rows_per_w, seq, d_emb, chunks):
  wid = lax.axis_index("s") * _NUM_CORES + lax.axis_index("c")
  base = wid * rows_per_w
  b = base // seq
  s0 = base % seq

  pltpu.sync_copy(len_hbm.at[b], len_v)
  len_vec = len_v[...]
  len_scalar = len_vec[0]
  groups = d_emb // _LANES

  def chunk_body(g, _):
    row0 = base + g * _CHUNK
    pos0 = s0 + g * _CHUNK + 1
    pltpu.sync_copy(x_hbm.at[pl.ds(row0, _CHUNK)], xbuf)

    @pl.when(pos0 <= len_scalar)
    def _add_pe():
      for j in range(_CHUNK // _LANES):
        pos = pos0 + j * _LANES + lax.iota(jnp.int32, _LANES)
        idx = jnp.where(pos <= len_vec, pos, 0)
        idx_v[pl.ds(j * _LANES, _LANES)] = idx
      pltpu.async_copy(pe_hbm.at[idx_v], pbuf, sem).wait()

      def row_body(r, _):
        for j in range(groups):
          plsc.addupdate(xbuf.at[r, pl.ds(j * _LANES, _LANES)],
                         pbuf[r, pl.ds(j * _LANES, _LANES)])
        return 0

      lax.fori_loop(0, _CHUNK, row_body, 0)

    pltpu.sync_copy(xbuf, out_hbm.at[pl.ds(row0, _CHUNK)])
    return 0

  lax.fori_loop(0, chunks, chunk_body, 0)


def kernel(x, lengths, pe_weight):
  n_batch, n_seq, d_emb = x.shape
  total_rows = n_batch * n_seq
  rows_per_w = total_rows // _NW
  chunks = rows_per_w // _CHUNK

  xf = x.reshape(total_rows, d_emb)
  lens16 = jnp.broadcast_to(
      lengths.astype(jnp.int32)[:, None], (n_batch, _LANES))

  mesh = plsc.VectorSubcoreMesh(core_axis_name="c", subcore_axis_name="s")
  body = functools.partial(
      _pe_add_body, rows_per_w=rows_per_w, seq=n_seq, d_emb=d_emb,
      chunks=chunks)
  out = pl.kernel(
      body,
      out_type=jax.ShapeDtypeStruct((total_rows, d_emb), jnp.float32),
      mesh=mesh,
      scratch_types=[
          pltpu.VMEM((_LANES,), jnp.int32),
          pltpu.VMEM((_CHUNK,), jnp.int32),
          pltpu.VMEM((_CHUNK, d_emb), jnp.float32),
          pltpu.VMEM((_CHUNK, d_emb), jnp.float32),
          pltpu.SemaphoreType.DMA,
      ],
  )(xf, lens16, pe_weight)
  return out.reshape(n_batch, n_seq, d_emb)


# speedup vs baseline: 1.6910x; 1.6526x over previous
import functools

import jax
import jax.numpy as jnp
from jax import lax
from jax.experimental import pallas as pl
from jax.experimental.pallas import tpu as pltpu
from jax.experimental.pallas import tpu_sc as plsc

_NUM_CORES = 2
_NUM_SUBCORES = 16
_NW = _NUM_CORES * _NUM_SUBCORES
_CHUNK = 16
_NBUF = 4
_NPB = 2
_LANES = 16


def _pe_add_body(x_hbm, len_hbm, pe_hbm, out_hbm, len_v,
                 xs0, xs1, xs2, xs3, pb0, pb1, idx0, idx1, semx, semp, semo,
                 *, rows_per_w, seq, d_emb, chunks):
  wid = lax.axis_index("s") * _NUM_CORES + lax.axis_index("c")
  base = wid * rows_per_w
  b = base // seq
  s0 = base % seq
  xs = [xs0, xs1, xs2, xs3]
  pb = [pb0, pb1]
  idxv = [idx0, idx1]
  groups = d_emb // _LANES

  pltpu.sync_copy(len_hbm.at[b], len_v)
  len_scalar = len_v[...][0]

  def x_copy(g, slot):
    return pltpu.make_async_copy(
        x_hbm.at[pl.ds(base + g * _CHUNK, _CHUNK)], xs[slot], semx.at[slot])

  def out_copy(g, slot):
    return pltpu.make_async_copy(
        xs[slot], out_hbm.at[pl.ds(base + g * _CHUNK, _CHUNK)], semo.at[slot])

  def pe_start(g, slot):
    idxv[slot][...] = s0 + g * _CHUNK + 1 + lax.iota(jnp.int32, _CHUNK)
    pltpu.make_async_copy(
        pe_hbm.at[idxv[slot]], pb[slot], semp.at[slot]).start()

  def pe_wait(slot):
    pltpu.make_async_copy(
        pe_hbm.at[idxv[slot]], pb[slot], semp.at[slot]).wait()

  def add_needed(g):
    return s0 + g * _CHUNK + 1 <= len_scalar

  def do_add(g, slot, pslot):
    pe_wait(pslot)
    nrows = jnp.minimum(len_scalar - (s0 + g * _CHUNK), _CHUNK)

    def row_body(r, _):
      for j in range(groups):
        plsc.addupdate(xs[slot].at[r, pl.ds(j * _LANES, _LANES)],
                       pb[pslot][r, pl.ds(j * _LANES, _LANES)])
      return 0

    lax.fori_loop(0, nrows, row_body, 0)

  x_copy(0, 0).start()
  x_copy(1, 1).start()

  @pl.when(add_needed(0))
  def _():
    pe_start(0, 0)

  def outer(i, _):
    for bb in range(_NBUF):
      g = i * _NBUF + bb
      nslot = (bb + 2) % _NBUF
      npslot = (bb + 1) % _NPB

      @pl.when(g >= 2)
      def _():
        out_copy(g - 2, nslot).wait()

      @pl.when(g + 2 < chunks)
      def _():
        x_copy(g + 2, nslot).start()

      @pl.when(jnp.logical_and(g + 1 < chunks, add_needed(g + 1)))
      def _():
        pe_start(g + 1, npslot)

      x_copy(g, bb).wait()

      @pl.when(add_needed(g))
      def _():
        do_add(g, bb, bb % _NPB)

      out_copy(g, bb).start()
    return 0

  lax.fori_loop(0, chunks // _NBUF, outer, 0)
  out_copy(chunks - 2, (chunks - 2) % _NBUF).wait()
  out_copy(chunks - 1, (chunks - 1) % _NBUF).wait()


def kernel(x, lengths, pe_weight):
  n_batch, n_seq, d_emb = x.shape
  total_rows = n_batch * n_seq
  rows_per_w = total_rows // _NW
  chunks = rows_per_w // _CHUNK

  xf = x.reshape(total_rows, d_emb)
  lens16 = jnp.broadcast_to(
      lengths.astype(jnp.int32)[:, None], (n_batch, _LANES))

  mesh = plsc.VectorSubcoreMesh(core_axis_name="c", subcore_axis_name="s")
  body = functools.partial(
      _pe_add_body, rows_per_w=rows_per_w, seq=n_seq, d_emb=d_emb,
      chunks=chunks)
  out = pl.kernel(
      body,
      out_type=jax.ShapeDtypeStruct((total_rows, d_emb), jnp.float32),
      mesh=mesh,
      scratch_types=[
          pltpu.VMEM((_LANES,), jnp.int32),
          pltpu.VMEM((_CHUNK, d_emb), jnp.float32),
          pltpu.VMEM((_CHUNK, d_emb), jnp.float32),
          pltpu.VMEM((_CHUNK, d_emb), jnp.float32),
          pltpu.VMEM((_CHUNK, d_emb), jnp.float32),
          pltpu.VMEM((_CHUNK, d_emb), jnp.float32),
          pltpu.VMEM((_CHUNK, d_emb), jnp.float32),
          pltpu.VMEM((_CHUNK,), jnp.int32),
          pltpu.VMEM((_CHUNK,), jnp.int32),
          pltpu.SemaphoreType.DMA((_NBUF,)),
          pltpu.SemaphoreType.DMA((_NPB,)),
          pltpu.SemaphoreType.DMA((_NBUF,)),
      ],
  )(xf, lens16, pe_weight)
  return out.reshape(n_batch, n_seq, d_emb)


# speedup vs baseline: 1.8905x; 1.1180x over previous
import functools

import jax
import jax.numpy as jnp
from jax import lax
from jax.experimental import pallas as pl
from jax.experimental.pallas import tpu as pltpu
from jax.experimental.pallas import tpu_sc as plsc

_NUM_CORES = 2
_NUM_SUBCORES = 16
_NW = _NUM_CORES * _NUM_SUBCORES
_CHUNK = 16
_NBUF = 4
_NPB = 2
_LANES = 16


def _pe_add_body(x_hbm, len_hbm, pe_hbm, out_hbm, len_v,
                 xs0, xs1, xs2, xs3, pb0, pb1, idx0, idx1, semx, semp, semo,
                 *, rows_per_w, seq, d_emb, chunks):
  wid = lax.axis_index("s") * _NUM_CORES + lax.axis_index("c")
  wpb = _NW // (rows_per_w * _NW // seq)
  b = wid // wpb
  c = wid % wpb
  xs = [xs0, xs1, xs2, xs3]
  pb = [pb0, pb1]
  idxv = [idx0, idx1]
  groups = d_emb // _LANES

  def s_off(g):
    return (g * wpb + c) * _CHUNK

  pltpu.sync_copy(len_hbm.at[b], len_v)
  len_scalar = len_v[...][0]

  def x_copy(g, slot):
    return pltpu.make_async_copy(
        x_hbm.at[pl.ds(b * seq + s_off(g), _CHUNK)], xs[slot], semx.at[slot])

  def out_copy(g, slot):
    return pltpu.make_async_copy(
        xs[slot], out_hbm.at[pl.ds(b * seq + s_off(g), _CHUNK)],
        semo.at[slot])

  def pe_start(g, slot):
    idxv[slot][...] = s_off(g) + 1 + lax.iota(jnp.int32, _CHUNK)
    pltpu.make_async_copy(
        pe_hbm.at[idxv[slot]], pb[slot], semp.at[slot]).start()

  def pe_wait(slot):
    pltpu.make_async_copy(
        pe_hbm.at[idxv[slot]], pb[slot], semp.at[slot]).wait()

  def add_needed(g):
    return s_off(g) + 1 <= len_scalar

  def do_add(g, slot, pslot):
    pe_wait(pslot)
    nrows = jnp.minimum(len_scalar - s_off(g), _CHUNK)

    def row_body(r, _):
      for j in range(groups):
        plsc.addupdate(xs[slot].at[r, pl.ds(j * _LANES, _LANES)],
                       pb[pslot][r, pl.ds(j * _LANES, _LANES)])
      return 0

    lax.fori_loop(0, nrows, row_body, 0)

  x_copy(0, 0).start()
  x_copy(1, 1).start()

  @pl.when(add_needed(0))
  def _():
    pe_start(0, 0)

  def outer(i, _):
    for bb in range(_NBUF):
      g = i * _NBUF + bb
      nslot = (bb + 2) % _NBUF
      npslot = (bb + 1) % _NPB

      @pl.when(g >= 2)
      def _():
        out_copy(g - 2, nslot).wait()

      @pl.when(g + 2 < chunks)
      def _():
        x_copy(g + 2, nslot).start()

      @pl.when(jnp.logical_and(g + 1 < chunks, add_needed(g + 1)))
      def _():
        pe_start(g + 1, npslot)

      x_copy(g, bb).wait()

      @pl.when(add_needed(g))
      def _():
        do_add(g, bb, bb % _NPB)

      out_copy(g, bb).start()
    return 0

  lax.fori_loop(0, chunks // _NBUF, outer, 0)
  out_copy(chunks - 2, (chunks - 2) % _NBUF).wait()
  out_copy(chunks - 1, (chunks - 1) % _NBUF).wait()


def kernel(x, lengths, pe_weight):
  n_batch, n_seq, d_emb = x.shape
  total_rows = n_batch * n_seq
  rows_per_w = total_rows // _NW
  chunks = rows_per_w // _CHUNK

  xf = x.reshape(total_rows, d_emb)
  lens16 = jnp.broadcast_to(
      lengths.astype(jnp.int32)[:, None], (n_batch, _LANES))

  mesh = plsc.VectorSubcoreMesh(core_axis_name="c", subcore_axis_name="s")
  body = functools.partial(
      _pe_add_body, rows_per_w=rows_per_w, seq=n_seq, d_emb=d_emb,
      chunks=chunks)
  out = pl.kernel(
      body,
      out_type=jax.ShapeDtypeStruct((total_rows, d_emb), jnp.float32),
      mesh=mesh,
      scratch_types=[
          pltpu.VMEM((_LANES,), jnp.int32),
          pltpu.VMEM((_CHUNK, d_emb), jnp.float32),
          pltpu.VMEM((_CHUNK, d_emb), jnp.float32),
          pltpu.VMEM((_CHUNK, d_emb), jnp.float32),
          pltpu.VMEM((_CHUNK, d_emb), jnp.float32),
          pltpu.VMEM((_CHUNK, d_emb), jnp.float32),
          pltpu.VMEM((_CHUNK, d_emb), jnp.float32),
          pltpu.VMEM((_CHUNK,), jnp.int32),
          pltpu.VMEM((_CHUNK,), jnp.int32),
          pltpu.SemaphoreType.DMA((_NBUF,)),
          pltpu.SemaphoreType.DMA((_NPB,)),
          pltpu.SemaphoreType.DMA((_NBUF,)),
      ],
  )(xf, lens16, pe_weight)
  return out.reshape(n_batch, n_seq, d_emb)
